# Initial kernel scaffold; baseline (speedup 1.0000x reference)
import jax
import jax.numpy as jnp
from jax.experimental import pallas as pl


def kernel(x, table):
    raise NotImplementedError("write your pallas kernel here")


# speedup vs baseline: 1.0158x; 1.0158x over previous
import functools

import jax
import jax.numpy as jnp
from jax import lax
from jax.experimental import pallas as pl
from jax.experimental.pallas import tpu as pltpu
from jax.experimental.pallas import tpu_sc as plsc

_D = 1000
_NC, _NS = 2, 16
_NW = _NC * _NS
_B = 1024 * 50
_BPW = _B // _NW
_CHUNK = 64
_NCHUNK = _BPW // _CHUNK


def _gather_rows(table, idx):
  mesh = plsc.VectorSubcoreMesh(core_axis_name="c", subcore_axis_name="s")

  @functools.partial(
      pl.kernel,
      mesh=mesh,
      compiler_params=pltpu.CompilerParams(use_tc_tiling_on_sc=False),
      out_type=jax.ShapeDtypeStruct((_B, _D), jnp.float32),
      scratch_types=[
          pltpu.VMEM((_BPW,), jnp.int32),
          pltpu.VMEM((2, _CHUNK, _D), jnp.float32),
          pltpu.SemaphoreType.DMA,
      ],
  )
  def body(table_hbm, idx_hbm, out_hbm, idx_v, rows_v, gsem):
    wid = lax.axis_index("s") * _NC + lax.axis_index("c")
    base = pl.multiple_of(wid * _BPW, 8)
    pltpu.sync_copy(idx_hbm.at[pl.ds(base, _BPW)], idx_v)

    def chunk(i, carry):
      off = pl.multiple_of(i * _CHUNK, 8)
      pltpu.async_copy(
          table_hbm.at[idx_v.at[pl.ds(off, _CHUNK)]], rows_v.at[0], gsem
      ).wait()
      pltpu.sync_copy(rows_v.at[0], out_hbm.at[pl.ds(base + off, _CHUNK)])
      return carry

    lax.fori_loop(0, _NCHUNK, chunk, 0)

  return body(table, idx)


def kernel(x, table):
  idx = x.reshape(-1).astype(jnp.int32)
  out = _gather_rows(table, idx)
  return out.reshape(x.shape[0], x.shape[1], _D)


# speedup vs baseline: 1.0332x; 1.0172x over previous
import functools

import jax
import jax.numpy as jnp
from jax import lax
from jax.experimental import pallas as pl
from jax.experimental.pallas import tpu as pltpu
from jax.experimental.pallas import tpu_sc as plsc

_D = 1000
_NC, _NS = 2, 16
_NW = _NC * _NS
_B = 1024 * 50
_BPW = _B // _NW
_CHUNK = 64
_NCHUNK = _BPW // _CHUNK


def _gather_rows(table, idx):
  mesh = plsc.VectorSubcoreMesh(core_axis_name="c", subcore_axis_name="s")

  @functools.partial(
      pl.kernel,
      mesh=mesh,
      compiler_params=pltpu.CompilerParams(use_tc_tiling_on_sc=False),
      out_type=jax.ShapeDtypeStruct((_B, _D), jnp.float32),
      scratch_types=[
          pltpu.VMEM((_BPW,), jnp.int32),
          pltpu.VMEM((2, _CHUNK, _D), jnp.float32),
          pltpu.SemaphoreType.DMA((2,)),
          pltpu.SemaphoreType.DMA((2,)),
      ],
  )
  def body(table_hbm, idx_hbm, out_hbm, idx_v, rows_v, gsem, ssem):
    wid = lax.axis_index("s") * _NC + lax.axis_index("c")
    base = pl.multiple_of(wid * _BPW, 8)
    pltpu.sync_copy(idx_hbm.at[pl.ds(base, _BPW)], idx_v)

    def gather_desc(c, b):
      off = pl.multiple_of(c * _CHUNK, 8)
      return pltpu.make_async_copy(
          table_hbm.at[idx_v.at[pl.ds(off, _CHUNK)]], rows_v.at[b], gsem.at[b]
      )

    def scatter_desc(c, b):
      off = pl.multiple_of(c * _CHUNK, 8)
      return pltpu.make_async_copy(
          rows_v.at[b], out_hbm.at[pl.ds(base + off, _CHUNK)], ssem.at[b]
      )

    gather_desc(0, 0).start()
    gather_desc(1, 1).start()

    def chunk(c, carry):
      b = lax.rem(c, 2)
      gather_desc(c, b).wait()
      scatter_desc(c, b).start()

      @pl.when(c + 2 < _NCHUNK)
      def _():
        scatter_desc(c, b).wait()
        gather_desc(c + 2, b).start()

      return carry

    lax.fori_loop(0, _NCHUNK, chunk, 0)
    scatter_desc(_NCHUNK - 2, lax.rem(_NCHUNK - 2, 2)).wait()
    scatter_desc(_NCHUNK - 1, lax.rem(_NCHUNK - 1, 2)).wait()

  return body(table, idx)


def kernel(x, table):
  idx = x.reshape(-1).astype(jnp.int32)
  out = _gather_rows(table, idx)
  return out.reshape(x.shape[0], x.shape[1], _D)


# speedup vs baseline: 1.0378x; 1.0045x over previous
import functools

import jax
import jax.numpy as jnp
from jax import lax
from jax.experimental import pallas as pl
from jax.experimental.pallas import tpu as pltpu
from jax.experimental.pallas import tpu_sc as plsc

_D = 1000
_NC, _NS = 2, 16
_NW = _NC * _NS
_BATCH = 1024
_SEQ = 50
_BPW = _BATCH // _NW


def _gather_rows(table, x):
  mesh = plsc.VectorSubcoreMesh(core_axis_name="c", subcore_axis_name="s")

  @functools.partial(
      pl.kernel,
      mesh=mesh,
      compiler_params=pltpu.CompilerParams(use_tc_tiling_on_sc=False),
      out_type=jax.ShapeDtypeStruct((_BATCH, _SEQ, _D), jnp.float32),
      scratch_types=[
          pltpu.VMEM((_BPW, _SEQ), jnp.int32),
          pltpu.VMEM((2, _SEQ, _D), jnp.float32),
          pltpu.SemaphoreType.DMA((2,)),
          pltpu.SemaphoreType.DMA((2,)),
      ],
  )
  def body(table_hbm, x_hbm, out_hbm, idx_v, rows_v, gsem, ssem):
    wid = lax.axis_index("s") * _NC + lax.axis_index("c")
    base = pl.multiple_of(wid * _BPW, 8)
    pltpu.sync_copy(x_hbm.at[pl.ds(base, _BPW)], idx_v)

    def gather_desc(c, b):
      return pltpu.make_async_copy(
          table_hbm.at[idx_v.at[c]], rows_v.at[b], gsem.at[b]
      )

    def scatter_desc(c, b):
      return pltpu.make_async_copy(
          rows_v.at[b], out_hbm.at[base + c], ssem.at[b]
      )

    gather_desc(0, 0).start()
    gather_desc(1, 1).start()

    def chunk(c, carry):
      b = lax.rem(c, 2)
      gather_desc(c, b).wait()
      scatter_desc(c, b).start()

      @pl.when(c + 2 < _BPW)
      def _():
        scatter_desc(c, b).wait()
        gather_desc(c + 2, b).start()

      return carry

    lax.fori_loop(0, _BPW, chunk, 0)
    scatter_desc(_BPW - 2, 0).wait()
    scatter_desc(_BPW - 1, 1).wait()

  return body(table, x)


def kernel(x, table):
  return _gather_rows(table, x.astype(jnp.int32))


# speedup vs baseline: 1.9832x; 1.9109x over previous
import functools

import jax
import jax.numpy as jnp
from jax import lax
from jax.experimental import pallas as pl
from jax.experimental.pallas import tpu as pltpu
from jax.experimental.pallas import tpu_sc as plsc

_D = 1000
_DP = 1024
_NC, _NS = 2, 16
_NW = _NC * _NS
_BATCH = 1024
_SEQ = 50
_SEQP = 56
_LSTR = 64
_BPW = _BATCH // _NW
_LPW = _BPW * _LSTR


def _gather_rows(table_p, ilist):
  mesh = plsc.VectorSubcoreMesh(core_axis_name="c", subcore_axis_name="s")

  @functools.partial(
      pl.kernel,
      mesh=mesh,
      out_type=jax.ShapeDtypeStruct((_BATCH, _SEQP, _DP), jnp.float32),
      scratch_types=[
          pltpu.VMEM((_LPW,), jnp.int32),
          pltpu.VMEM((2, _SEQP, _DP), jnp.float32),
          pltpu.SemaphoreType.DMA((2,)),
          pltpu.SemaphoreType.DMA((2,)),
      ],
  )
  def body(table_hbm, ilist_hbm, out_hbm, ilist_v, rows_v, gsem, ssem):
    wid = lax.axis_index("s") * _NC + lax.axis_index("c")
    base = pl.multiple_of(wid * _LPW, 8)
    pltpu.sync_copy(ilist_hbm.at[pl.ds(base, _LPW)], ilist_v)

    def gather_desc(j, b):
      off = pl.multiple_of(j * _LSTR, 8)
      return pltpu.make_async_copy(
          table_hbm.at[ilist_v.at[pl.ds(off, _SEQP)]], rows_v.at[b], gsem.at[b]
      )

    def scatter_desc(j, b):
      return pltpu.make_async_copy(
          rows_v.at[b], out_hbm.at[wid * _BPW + j], ssem.at[b]
      )

    gather_desc(0, 0).start()
    gather_desc(1, 1).start()

    def batch_step(j, carry):
      b = lax.rem(j, 2)
      gather_desc(j, b).wait()
      scatter_desc(j, b).start()

      @pl.when(j + 2 < _BPW)
      def _():
        scatter_desc(j, b).wait()
        gather_desc(j + 2, b).start()

      return carry

    lax.fori_loop(0, _BPW, batch_step, 0)
    scatter_desc(_BPW - 2, 0).wait()
    scatter_desc(_BPW - 1, 1).wait()

  return body(table_p, ilist)


def kernel(x, table):
  xi = x.astype(jnp.int32)
  ilist = jnp.concatenate(
      [xi, jnp.broadcast_to(xi[:, _SEQ - 1 :], (_BATCH, _LSTR - _SEQ))], axis=1
  ).reshape(-1)
  table_p = jnp.pad(table, ((0, 0), (0, _DP - _D)))
  padded = _gather_rows(table_p, ilist)
  return padded[:, :_SEQ, :_D]
